# Initial kernel scaffold; baseline (speedup 1.0000x reference)
import jax
import jax.numpy as jnp
from jax.experimental import pallas as pl


def kernel(x, image_feature, adj1, adj2, W_enc1, b_enc1, g_enc1, be_enc1, W_enc2, b_enc2, g_enc2, be_enc2, W_gc1, W_gc2, W_gc3, W_dec, w_omega, u_omega, W_rna, b_rna, W_img, b_img, cluster, enc_mask_token):
    raise NotImplementedError("write your pallas kernel here")


# speedup vs baseline: 1.2567x; 1.2567x over previous
import functools

import jax
import jax.numpy as jnp
from jax.experimental import pallas as pl
from jax.experimental.pallas import tpu as pltpu

F32 = jnp.float32
BF16 = jnp.bfloat16

_HI = jax.lax.Precision.HIGHEST
_INV_STD = float(1.0 / (1.0 + 0.001) ** 0.5)

_BR = 400
_BRS = 2000


def _elu(v):
    return jnp.where(v > 0, v, jnp.exp(v) - 1.0)


def _enc_body(x_ref, img_ref, c_ref, tok_ref, W1_ref, s1_ref, o1_ref,
              W2_ref, s2_ref, o2_ref, Wg1_ref, Wimg_ref, bimg_ref,
              feat_ref, P_ref, iemb_ref):
    xe = x_ref[...] + c_ref[...] * tok_ref[...]
    h = jax.lax.dot_general(xe, W1_ref[...], (((1,), (0,)), ((), ())),
                            precision=_HI)
    h = _elu(h * s1_ref[...] + o1_ref[...])
    f = jax.lax.dot_general(h, W2_ref[...], (((1,), (0,)), ((), ())),
                            precision=_HI)
    f = _elu(f * s2_ref[...] + o2_ref[...])
    feat_ref[...] = f
    P = jax.lax.dot_general(f, Wg1_ref[...], (((1,), (0,)), ((), ())),
                            precision=_HI)
    P_ref[...] = P.astype(BF16)
    ie = jax.lax.dot_general(img_ref[...], Wimg_ref[...],
                             (((1,), (0,)), ((), ())), precision=_HI)
    iemb_ref[...] = ie + bimg_ref[...]


def _p1_body(adj_ref, P_ref, Wg23_ref, S_ref, abf_ref):
    ab = adj_ref[...].astype(BF16)
    abf_ref[...] = ab
    t = jax.lax.dot_general(ab, P_ref[...], (((1,), (0,)), ((), ())),
                            preferred_element_type=F32)
    h = jnp.maximum(t, 0.0)
    s = jax.lax.dot_general(h, Wg23_ref[...], (((1,), (0,)), ((), ())),
                            precision=_HI)
    S_ref[...] = s.astype(BF16)


def _p2_body(abf_ref, S_ref, ml_ref):
    ml_ref[...] = jax.lax.dot_general(
        abf_ref[...], S_ref[...], (((1,), (0,)), ((), ())),
        preferred_element_type=F32)


def _mid_body(f_ref, mu1_ref, mu2_ref, w_ref, u_ref, Wr_ref, br_ref,
              Wd_ref, cT_ref, gnn_ref, z_ref, rna_ref, dec_ref, q_ref):
    mu1 = mu1_ref[...]
    mu2 = mu2_ref[...]
    v1 = jnp.tanh(jax.lax.dot_general(mu1, w_ref[...], (((1,), (0,)), ((), ())),
                                      precision=_HI))
    v2 = jnp.tanh(jax.lax.dot_general(mu2, w_ref[...], (((1,), (0,)), ((), ())),
                                      precision=_HI))
    s1 = jax.lax.dot_general(v1, u_ref[...], (((1,), (0,)), ((), ())),
                             precision=_HI)
    s2 = jax.lax.dot_general(v2, u_ref[...], (((1,), (0,)), ((), ())),
                             precision=_HI)
    w1 = jax.nn.sigmoid(s1 - s2)
    gnn = w1 * mu1 + (1.0 - w1) * mu2
    gnn_ref[...] = gnn
    z = jnp.concatenate([f_ref[...], gnn], axis=1)
    z_ref[...] = z
    rna = jax.lax.dot_general(z, Wr_ref[...], (((1,), (0,)), ((), ())),
                              precision=_HI) + br_ref[...]
    rna_ref[...] = rna
    dec = jax.lax.dot_general(rna, Wd_ref[...], (((1,), (0,)), ((), ())),
                              precision=_HI)
    dec_ref[...] = dec.astype(BF16)
    cT = cT_ref[...]
    zn2 = jnp.sum(z * z, axis=1, keepdims=True)
    cn2 = jnp.sum(cT * cT, axis=0, keepdims=True)
    cross = jax.lax.dot_general(z, cT, (((1,), (0,)), ((), ())),
                                precision=_HI)
    qd = 1.0 / (1.0 + (zn2 - 2.0 * cross + cn2))
    q_ref[...] = qd / jnp.sum(qd, axis=1, keepdims=True)


def _p3_body(abf_ref, dec_ref, x_ref, c_ref, tok_ref, m2_ref,
             de_ref, lp_ref):
    de = jax.lax.dot_general(abf_ref[...], dec_ref[...],
                             (((1,), (0,)), ((), ())),
                             preferred_element_type=F32)
    de_ref[...] = de
    xe = x_ref[...] + c_ref[...] * tok_ref[...]
    dn = jnp.maximum(jnp.sqrt(jnp.sum(de * de, axis=1, keepdims=True)), 1e-12)
    xn = jnp.maximum(jnp.sqrt(jnp.sum(xe * xe, axis=1, keepdims=True)), 1e-12)
    cos = jnp.sum(de * xe, axis=1, keepdims=True) / (dn * xn)
    t = 1.0 - cos
    s = jnp.sum(m2_ref[...] * t * t * t)
    lp_ref[...] = jnp.broadcast_to(s, (1, 1, 128))


def _full_spec(shape):
    nd = len(shape)
    return pl.BlockSpec(shape, lambda i: (0,) * nd)


def _row_spec(br, ncols):
    return pl.BlockSpec((br, ncols), lambda i: (i, 0))


_PARAMS = pltpu.CompilerParams(dimension_semantics=("parallel",))


def kernel(x, image_feature, adj1, adj2, W_enc1, b_enc1, g_enc1, be_enc1,
           W_enc2, b_enc2, g_enc2, be_enc2, W_gc1, W_gc2, W_gc3, W_dec,
           w_omega, u_omega, W_rna, b_rna, W_img, b_img, cluster,
           enc_mask_token):
    n, d = x.shape
    img_d = image_feature.shape[1]
    num_mask = int(0.8 * n)

    perm1 = jax.random.permutation(jax.random.key(1), n)
    perm2 = jax.random.permutation(jax.random.key(2), n)
    m1 = jnp.zeros((n,), F32).at[perm1[:num_mask]].set(1.0)
    m2 = jnp.zeros((n,), F32).at[perm2[:num_mask]].set(1.0)
    c = (m1 + m2).reshape(n, 1)
    m2 = m2.reshape(n, 1)
    tok = enc_mask_token.reshape(1, d)

    s1 = (_INV_STD * g_enc1).reshape(1, -1)
    o1 = (b_enc1 * _INV_STD * g_enc1 + be_enc1).reshape(1, -1)
    s2 = (_INV_STD * g_enc2).reshape(1, -1)
    o2 = (b_enc2 * _INV_STD * g_enc2 + be_enc2).reshape(1, -1)
    Wg23 = jnp.concatenate([W_gc2, W_gc3], axis=1)
    cT = cluster.T
    nc = cluster.shape[0]

    grid_s = (n // _BRS,)
    feat_x, P_bf, image_emb = pl.pallas_call(
        _enc_body,
        grid=grid_s,
        in_specs=[
            _row_spec(_BRS, d), _row_spec(_BRS, img_d), _row_spec(_BRS, 1),
            _full_spec((1, d)), _full_spec(W_enc1.shape), _full_spec(s1.shape),
            _full_spec(o1.shape), _full_spec(W_enc2.shape),
            _full_spec(s2.shape), _full_spec(o2.shape),
            _full_spec(W_gc1.shape), _full_spec(W_img.shape),
            _full_spec((1, W_img.shape[1])),
        ],
        out_specs=[_row_spec(_BRS, 16), _row_spec(_BRS, 64),
                   _row_spec(_BRS, 32)],
        out_shape=[jax.ShapeDtypeStruct((n, 16), F32),
                   jax.ShapeDtypeStruct((n, 64), BF16),
                   jax.ShapeDtypeStruct((n, 32), F32)],
        compiler_params=_PARAMS,
    )(x, image_feature, c, tok, W_enc1, s1, o1, W_enc2, s2, o2, W_gc1,
      W_img, b_img.reshape(1, -1))

    grid_a = (n // _BR,)

    def pass1(adj):
        return pl.pallas_call(
            _p1_body,
            grid=grid_a,
            in_specs=[_row_spec(_BR, n), _full_spec((n, 64)),
                      _full_spec((64, 32))],
            out_specs=[_row_spec(_BR, 32), _row_spec(_BR, n)],
            out_shape=[jax.ShapeDtypeStruct((n, 32), BF16),
                       jax.ShapeDtypeStruct((n, n), BF16)],
            compiler_params=_PARAMS,
        )(adj, P_bf, Wg23)

    S1, A1 = pass1(adj1)
    S2, A2 = pass1(adj2)

    def pass2(abf, S):
        return pl.pallas_call(
            _p2_body,
            grid=grid_a,
            in_specs=[_row_spec(_BR, n), _full_spec((n, 32))],
            out_specs=_row_spec(_BR, 32),
            out_shape=jax.ShapeDtypeStruct((n, 32), F32),
            compiler_params=_PARAMS,
        )(abf, S)

    ML1 = pass2(A1, S1)
    ML2 = pass2(A2, S2)
    mu1, logvar1 = ML1[:, :16], ML1[:, 16:]
    mu2, logvar2 = ML2[:, :16], ML2[:, 16:]

    gnn_z, z, rna_emb, dec_bf, q = pl.pallas_call(
        _mid_body,
        grid=grid_s,
        in_specs=[
            _row_spec(_BRS, 16), _row_spec(_BRS, 16), _row_spec(_BRS, 16),
            _full_spec(w_omega.shape), _full_spec(u_omega.shape),
            _full_spec(W_rna.shape), _full_spec((1, W_rna.shape[1])),
            _full_spec(W_dec.shape), _full_spec(cT.shape),
        ],
        out_specs=[_row_spec(_BRS, 16), _row_spec(_BRS, 32),
                   _row_spec(_BRS, 32), _row_spec(_BRS, d),
                   _row_spec(_BRS, nc)],
        out_shape=[jax.ShapeDtypeStruct((n, 16), F32),
                   jax.ShapeDtypeStruct((n, 32), F32),
                   jax.ShapeDtypeStruct((n, 32), F32),
                   jax.ShapeDtypeStruct((n, d), BF16),
                   jax.ShapeDtypeStruct((n, nc), F32)],
        compiler_params=_PARAMS,
    )(feat_x, mu1, mu2, w_omega, u_omega, W_rna, b_rna.reshape(1, -1),
      W_dec, cT)

    def pass3(abf):
        return pl.pallas_call(
            _p3_body,
            grid=grid_a,
            in_specs=[_row_spec(_BR, n), _full_spec((n, d)),
                      _row_spec(_BR, d), _row_spec(_BR, 1),
                      _full_spec((1, d)), _row_spec(_BR, 1)],
            out_specs=[_row_spec(_BR, d),
                       pl.BlockSpec((1, 1, 128), lambda i: (i, 0, 0))],
            out_shape=[jax.ShapeDtypeStruct((n, d), F32),
                       jax.ShapeDtypeStruct((n // _BR, 1, 128), F32)],
            compiler_params=_PARAMS,
        )(abf, dec_bf, x, c, tok, m2)

    de_feat1, lp1 = pass3(A1)
    de_feat2, lp2 = pass3(A2)
    loss = ((jnp.sum(lp1[:, 0, 0]) + jnp.sum(lp2[:, 0, 0])) / num_mask).astype(F32)

    return (z, mu1, logvar1, mu2, logvar2, de_feat1, de_feat2, q,
            feat_x, gnn_z, loss, rna_emb, image_emb)


# speedup vs baseline: 1.6650x; 1.3248x over previous
import functools

import numpy as np

import jax
import jax.numpy as jnp
from jax.experimental import pallas as pl
from jax.experimental.pallas import tpu as pltpu

F32 = jnp.float32
BF16 = jnp.bfloat16

_HI = jax.lax.Precision.HIGHEST
_INV_STD = float(1.0 / (1.0 + 0.001) ** 0.5)

_BR = 400
_BRS = 2000


@functools.lru_cache(maxsize=None)
def _mask_vectors(n):
    num_mask = int(0.8 * n)
    with jax.ensure_compile_time_eval():
        p1 = np.asarray(jax.random.permutation(jax.random.key(1), n))
        p2 = np.asarray(jax.random.permutation(jax.random.key(2), n))
    m1 = np.zeros((n, 1), np.float32)
    m2 = np.zeros((n, 1), np.float32)
    m1[p1[:num_mask]] = 1.0
    m2[p2[:num_mask]] = 1.0
    return m1, m2


def _dot(a, b, precision=None):
    return jax.lax.dot_general(a, b, (((1,), (0,)), ((), ())),
                               preferred_element_type=F32,
                               precision=precision)


def _elu(v):
    return jnp.where(v > 0, v, jnp.exp(v) - 1.0)


def _enc_body(x_ref, img_ref, c_ref, tok_ref, W1_ref, s1_ref, o1_ref,
              W2_ref, s2_ref, o2_ref, Wg1_ref, Wimg_ref, bimg_ref,
              feat_ref, P_ref, iemb_ref):
    xe = x_ref[...] + c_ref[...] * tok_ref[...]
    h = _dot(xe.astype(BF16), W1_ref[...])
    h = _elu(h * s1_ref[...] + o1_ref[...])
    f = _dot(h, W2_ref[...], precision=_HI)
    f = _elu(f * s2_ref[...] + o2_ref[...])
    feat_ref[...] = f
    P_ref[...] = _dot(f, Wg1_ref[...], precision=_HI).astype(BF16)
    iemb_ref[...] = _dot(img_ref[...].astype(BF16), Wimg_ref[...]) + bimg_ref[...]


def _p1_body(adj_ref, P_ref, Wg23_ref, S_ref, abf_ref):
    ab = adj_ref[...].astype(BF16)
    abf_ref[...] = ab
    t = _dot(ab, P_ref[...])
    h = jnp.maximum(t, 0.0)
    S_ref[...] = _dot(h.astype(BF16), Wg23_ref[...]).astype(BF16)


def _p2_body(a1_ref, a2_ref, S1_ref, S2_ref, f_ref, w_ref, u_ref,
             Wr_ref, br_ref, Wd_ref, cT_ref,
             mu1_ref, lv1_ref, mu2_ref, lv2_ref,
             gnn_ref, z_ref, rna_ref, dec_ref, q_ref):
    ml1 = _dot(a1_ref[...], S1_ref[...])
    ml2 = _dot(a2_ref[...], S2_ref[...])
    mu1 = ml1[:, :16]
    mu2 = ml2[:, :16]
    mu1_ref[...] = mu1
    lv1_ref[...] = ml1[:, 16:]
    mu2_ref[...] = mu2
    lv2_ref[...] = ml2[:, 16:]
    v1 = jnp.tanh(_dot(mu1, w_ref[...], precision=_HI))
    v2 = jnp.tanh(_dot(mu2, w_ref[...], precision=_HI))
    s1 = _dot(v1, u_ref[...], precision=_HI)
    s2 = _dot(v2, u_ref[...], precision=_HI)
    w1 = jax.nn.sigmoid(s1 - s2)
    gnn = w1 * mu1 + (1.0 - w1) * mu2
    gnn_ref[...] = gnn
    z = jnp.concatenate([f_ref[...], gnn], axis=1)
    z_ref[...] = z
    rna = _dot(z, Wr_ref[...], precision=_HI) + br_ref[...]
    rna_ref[...] = rna
    dec_ref[...] = _dot(rna, Wd_ref[...], precision=_HI).astype(BF16)
    cT = cT_ref[...]
    zn2 = jnp.sum(z * z, axis=1, keepdims=True)
    cn2 = jnp.sum(cT * cT, axis=0, keepdims=True)
    cross = _dot(z, cT, precision=_HI)
    qd = 1.0 / (1.0 + (zn2 - 2.0 * cross + cn2))
    q_ref[...] = qd / jnp.sum(qd, axis=1, keepdims=True)


def _p3_body(a1_ref, a2_ref, dec_ref, x_ref, c_ref, tok_ref, m2_ref,
             de1_ref, de2_ref, lp1_ref, lp2_ref):
    dec = dec_ref[...]
    xe = x_ref[...] + c_ref[...] * tok_ref[...]
    xn = jnp.maximum(jnp.sqrt(jnp.sum(xe * xe, axis=1, keepdims=True)), 1e-12)
    m2 = m2_ref[...]

    def sce(de, lp_ref):
        dn = jnp.maximum(jnp.sqrt(jnp.sum(de * de, axis=1, keepdims=True)),
                         1e-12)
        cos = jnp.sum(de * xe, axis=1, keepdims=True) / (dn * xn)
        t = 1.0 - cos
        lp_ref[...] = jnp.broadcast_to(jnp.sum(m2 * t * t * t), (1, 1, 128))

    de1 = _dot(a1_ref[...], dec)
    de1_ref[...] = de1
    sce(de1, lp1_ref)
    de2 = _dot(a2_ref[...], dec)
    de2_ref[...] = de2
    sce(de2, lp2_ref)


def _full_spec(shape):
    nd = len(shape)
    return pl.BlockSpec(shape, lambda i: (0,) * nd)


def _row_spec(br, ncols):
    return pl.BlockSpec((br, ncols), lambda i: (i, 0))


_PARAMS = pltpu.CompilerParams(dimension_semantics=("parallel",))


def kernel(x, image_feature, adj1, adj2, W_enc1, b_enc1, g_enc1, be_enc1,
           W_enc2, b_enc2, g_enc2, be_enc2, W_gc1, W_gc2, W_gc3, W_dec,
           w_omega, u_omega, W_rna, b_rna, W_img, b_img, cluster,
           enc_mask_token):
    n, d = x.shape
    img_d = image_feature.shape[1]
    num_mask = int(0.8 * n)

    m1, m2 = _mask_vectors(n)
    c = jnp.asarray(m1 + m2)
    m2 = jnp.asarray(m2)
    tok = enc_mask_token.reshape(1, d)

    s1 = (_INV_STD * g_enc1).reshape(1, -1)
    o1 = (b_enc1 * _INV_STD * g_enc1 + be_enc1).reshape(1, -1)
    s2 = (_INV_STD * g_enc2).reshape(1, -1)
    o2 = (b_enc2 * _INV_STD * g_enc2 + be_enc2).reshape(1, -1)
    Wg23 = jnp.concatenate([W_gc2, W_gc3], axis=1).astype(BF16)
    cT = cluster.T
    nc = cluster.shape[0]

    grid_s = (n // _BRS,)
    feat_x, P_bf, image_emb = pl.pallas_call(
        _enc_body,
        grid=grid_s,
        in_specs=[
            _row_spec(_BRS, d), _row_spec(_BRS, img_d), _row_spec(_BRS, 1),
            _full_spec((1, d)), _full_spec(W_enc1.shape), _full_spec(s1.shape),
            _full_spec(o1.shape), _full_spec(W_enc2.shape),
            _full_spec(s2.shape), _full_spec(o2.shape),
            _full_spec(W_gc1.shape), _full_spec(W_img.shape),
            _full_spec((1, W_img.shape[1])),
        ],
        out_specs=[_row_spec(_BRS, 16), _row_spec(_BRS, 64),
                   _row_spec(_BRS, 32)],
        out_shape=[jax.ShapeDtypeStruct((n, 16), F32),
                   jax.ShapeDtypeStruct((n, 64), BF16),
                   jax.ShapeDtypeStruct((n, 32), F32)],
        compiler_params=_PARAMS,
    )(x, image_feature, c, tok, W_enc1.astype(BF16), s1, o1, W_enc2, s2, o2,
      W_gc1, W_img.astype(BF16), b_img.reshape(1, -1))

    grid_a = (n // _BR,)

    def pass1(adj):
        return pl.pallas_call(
            _p1_body,
            grid=grid_a,
            in_specs=[_row_spec(_BR, n), _full_spec((n, 64)),
                      _full_spec((64, 32))],
            out_specs=[_row_spec(_BR, 32), _row_spec(_BR, n)],
            out_shape=[jax.ShapeDtypeStruct((n, 32), BF16),
                       jax.ShapeDtypeStruct((n, n), BF16)],
            compiler_params=_PARAMS,
        )(adj, P_bf, Wg23)

    S1, A1 = pass1(adj1)
    S2, A2 = pass1(adj2)

    small16 = [_row_spec(_BR, 16), jax.ShapeDtypeStruct((n, 16), F32)]
    mu1, logvar1, mu2, logvar2, gnn_z, z, rna_emb, dec_bf, q = pl.pallas_call(
        _p2_body,
        grid=grid_a,
        in_specs=[
            _row_spec(_BR, n), _row_spec(_BR, n),
            _full_spec((n, 32)), _full_spec((n, 32)),
            _row_spec(_BR, 16),
            _full_spec(w_omega.shape), _full_spec(u_omega.shape),
            _full_spec(W_rna.shape), _full_spec((1, W_rna.shape[1])),
            _full_spec(W_dec.shape), _full_spec(cT.shape),
        ],
        out_specs=[small16[0], small16[0], small16[0], small16[0],
                   small16[0], _row_spec(_BR, 32), _row_spec(_BR, 32),
                   _row_spec(_BR, d), _row_spec(_BR, nc)],
        out_shape=[small16[1], small16[1], small16[1], small16[1],
                   small16[1],
                   jax.ShapeDtypeStruct((n, 32), F32),
                   jax.ShapeDtypeStruct((n, 32), F32),
                   jax.ShapeDtypeStruct((n, d), BF16),
                   jax.ShapeDtypeStruct((n, nc), F32)],
        compiler_params=_PARAMS,
    )(A1, A2, S1, S2, feat_x, w_omega, u_omega, W_rna,
      b_rna.reshape(1, -1), W_dec, cT)

    de_feat1, de_feat2, lp1, lp2 = pl.pallas_call(
        _p3_body,
        grid=grid_a,
        in_specs=[_row_spec(_BR, n), _row_spec(_BR, n), _full_spec((n, d)),
                  _row_spec(_BR, d), _row_spec(_BR, 1),
                  _full_spec((1, d)), _row_spec(_BR, 1)],
        out_specs=[_row_spec(_BR, d), _row_spec(_BR, d),
                   pl.BlockSpec((1, 1, 128), lambda i: (i, 0, 0)),
                   pl.BlockSpec((1, 1, 128), lambda i: (i, 0, 0))],
        out_shape=[jax.ShapeDtypeStruct((n, d), F32),
                   jax.ShapeDtypeStruct((n, d), F32),
                   jax.ShapeDtypeStruct((n // _BR, 1, 128), F32),
                   jax.ShapeDtypeStruct((n // _BR, 1, 128), F32)],
        compiler_params=_PARAMS,
    )(A1, A2, dec_bf, x, c, tok, m2)

    loss = ((jnp.sum(lp1[:, 0, 0]) + jnp.sum(lp2[:, 0, 0]))
            / num_mask).astype(F32)

    return (z, mu1, logvar1, mu2, logvar2, de_feat1, de_feat2, q,
            feat_x, gnn_z, loss, rna_emb, image_emb)


# speedup vs baseline: 1.6773x; 1.0074x over previous
import functools

import numpy as np

import jax
import jax.numpy as jnp
from jax.experimental import pallas as pl
from jax.experimental.pallas import tpu as pltpu

F32 = jnp.float32
BF16 = jnp.bfloat16

_HI = jax.lax.Precision.HIGHEST
_INV_STD = float(1.0 / (1.0 + 0.001) ** 0.5)

_BR = 400
_BRS = 2000


@functools.lru_cache(maxsize=None)
def _mask_vectors_host(n):
    num_mask = int(0.8 * n)
    with jax.ensure_compile_time_eval():
        p1 = np.asarray(jax.random.permutation(jax.random.key(1), n))
        p2 = np.asarray(jax.random.permutation(jax.random.key(2), n))
    m1 = np.zeros((n, 1), np.float32)
    m2 = np.zeros((n, 1), np.float32)
    m1[p1[:num_mask]] = 1.0
    m2[p2[:num_mask]] = 1.0
    return m1, m2


def _mask_vectors(n):
    num_mask = int(0.8 * n)
    try:
        return _mask_vectors_host(n)
    except Exception:
        p1 = jax.random.permutation(jax.random.key(1), n)
        p2 = jax.random.permutation(jax.random.key(2), n)
        m1 = jnp.zeros((n, 1), F32).at[p1[:num_mask], 0].set(1.0)
        m2 = jnp.zeros((n, 1), F32).at[p2[:num_mask], 0].set(1.0)
        return m1, m2


def _dot(a, b, precision=None):
    return jax.lax.dot_general(a, b, (((1,), (0,)), ((), ())),
                               preferred_element_type=F32,
                               precision=precision)


def _elu(v):
    return jnp.where(v > 0, v, jnp.exp(v) - 1.0)


def _enc_body(x_ref, img_ref, c_ref, tok_ref, W1_ref, s1_ref, o1_ref,
              W2_ref, s2_ref, o2_ref, Wg1_ref, Wimg_ref, bimg_ref,
              feat_ref, P_ref, iemb_ref):
    xe = x_ref[...] + c_ref[...] * tok_ref[...]
    h = _dot(xe.astype(BF16), W1_ref[...])
    h = _elu(h * s1_ref[...] + o1_ref[...])
    f = _dot(h.astype(BF16), W2_ref[...])
    f = _elu(f * s2_ref[...] + o2_ref[...])
    feat_ref[...] = f
    P_ref[...] = _dot(f.astype(BF16), Wg1_ref[...]).astype(BF16)
    iemb_ref[...] = _dot(img_ref[...].astype(BF16), Wimg_ref[...]) + bimg_ref[...]


def _p1_body(adj_ref, P_ref, Wg23_ref, S_ref, abf_ref):
    ab = adj_ref[...].astype(BF16)
    abf_ref[...] = ab
    t = _dot(ab, P_ref[...])
    h = jnp.maximum(t, 0.0)
    S_ref[...] = _dot(h.astype(BF16), Wg23_ref[...]).astype(BF16)


def _p2_body(a1_ref, a2_ref, S1_ref, S2_ref, f_ref, w_ref, u_ref,
             Wr_ref, br_ref, Wd_ref, cT_ref,
             mu1_ref, lv1_ref, mu2_ref, lv2_ref,
             gnn_ref, z_ref, rna_ref, dec_ref, q_ref):
    ml1 = _dot(a1_ref[...], S1_ref[...])
    ml2 = _dot(a2_ref[...], S2_ref[...])
    mu1 = ml1[:, :16]
    mu2 = ml2[:, :16]
    mu1_ref[...] = mu1
    lv1_ref[...] = ml1[:, 16:]
    mu2_ref[...] = mu2
    lv2_ref[...] = ml2[:, 16:]
    v1 = jnp.tanh(_dot(mu1, w_ref[...], precision=_HI))
    v2 = jnp.tanh(_dot(mu2, w_ref[...], precision=_HI))
    s1 = _dot(v1, u_ref[...], precision=_HI)
    s2 = _dot(v2, u_ref[...], precision=_HI)
    w1 = jax.nn.sigmoid(s1 - s2)
    gnn = w1 * mu1 + (1.0 - w1) * mu2
    gnn_ref[...] = gnn
    z = jnp.concatenate([f_ref[...], gnn], axis=1)
    z_ref[...] = z
    rna = _dot(z, Wr_ref[...], precision=_HI) + br_ref[...]
    rna_ref[...] = rna
    dec_ref[...] = _dot(rna, Wd_ref[...], precision=_HI).astype(BF16)
    cT = cT_ref[...]
    zn2 = jnp.sum(z * z, axis=1, keepdims=True)
    cn2 = jnp.sum(cT * cT, axis=0, keepdims=True)
    cross = _dot(z, cT, precision=_HI)
    qd = 1.0 / (1.0 + (zn2 - 2.0 * cross + cn2))
    q_ref[...] = qd / jnp.sum(qd, axis=1, keepdims=True)


def _p3_body(a1_ref, a2_ref, dec_ref, x_ref, c_ref, tok_ref, m2_ref,
             de1_ref, de2_ref, lp1_ref, lp2_ref):
    dec = dec_ref[...]
    xe = x_ref[...] + c_ref[...] * tok_ref[...]
    xn = jnp.maximum(jnp.sqrt(jnp.sum(xe * xe, axis=1, keepdims=True)), 1e-12)
    m2 = m2_ref[...]

    def sce(de, lp_ref):
        dn = jnp.maximum(jnp.sqrt(jnp.sum(de * de, axis=1, keepdims=True)),
                         1e-12)
        cos = jnp.sum(de * xe, axis=1, keepdims=True) / (dn * xn)
        t = 1.0 - cos
        lp_ref[...] = jnp.broadcast_to(jnp.sum(m2 * t * t * t), (1, 1, 128))

    de1 = _dot(a1_ref[...], dec)
    de1_ref[...] = de1
    sce(de1, lp1_ref)
    de2 = _dot(a2_ref[...], dec)
    de2_ref[...] = de2
    sce(de2, lp2_ref)


def _full_spec(shape):
    nd = len(shape)
    return pl.BlockSpec(shape, lambda i: (0,) * nd)


def _row_spec(br, ncols):
    return pl.BlockSpec((br, ncols), lambda i: (i, 0))


_PARAMS = pltpu.CompilerParams(dimension_semantics=("parallel",))


def kernel(x, image_feature, adj1, adj2, W_enc1, b_enc1, g_enc1, be_enc1,
           W_enc2, b_enc2, g_enc2, be_enc2, W_gc1, W_gc2, W_gc3, W_dec,
           w_omega, u_omega, W_rna, b_rna, W_img, b_img, cluster,
           enc_mask_token):
    n, d = x.shape
    img_d = image_feature.shape[1]
    num_mask = int(0.8 * n)

    m1, m2 = _mask_vectors(n)
    c = jnp.asarray(m1 + m2)
    m2 = jnp.asarray(m2)
    tok = enc_mask_token.reshape(1, d)

    s1 = (_INV_STD * g_enc1).reshape(1, -1)
    o1 = (b_enc1 * _INV_STD * g_enc1 + be_enc1).reshape(1, -1)
    s2 = (_INV_STD * g_enc2).reshape(1, -1)
    o2 = (b_enc2 * _INV_STD * g_enc2 + be_enc2).reshape(1, -1)
    Wg23 = jnp.concatenate([W_gc2, W_gc3], axis=1).astype(BF16)
    cT = cluster.T
    nc = cluster.shape[0]

    grid_s = (n // _BRS,)
    feat_x, P_bf, image_emb = pl.pallas_call(
        _enc_body,
        grid=grid_s,
        in_specs=[
            _row_spec(_BRS, d), _row_spec(_BRS, img_d), _row_spec(_BRS, 1),
            _full_spec((1, d)), _full_spec(W_enc1.shape), _full_spec(s1.shape),
            _full_spec(o1.shape), _full_spec(W_enc2.shape),
            _full_spec(s2.shape), _full_spec(o2.shape),
            _full_spec(W_gc1.shape), _full_spec(W_img.shape),
            _full_spec((1, W_img.shape[1])),
        ],
        out_specs=[_row_spec(_BRS, 16), _row_spec(_BRS, 64),
                   _row_spec(_BRS, 32)],
        out_shape=[jax.ShapeDtypeStruct((n, 16), F32),
                   jax.ShapeDtypeStruct((n, 64), BF16),
                   jax.ShapeDtypeStruct((n, 32), F32)],
        compiler_params=_PARAMS,
    )(x, image_feature, c, tok, W_enc1.astype(BF16), s1, o1,
      W_enc2.astype(BF16), s2, o2, W_gc1.astype(BF16),
      W_img.astype(BF16), b_img.reshape(1, -1))

    grid_a = (n // _BR,)

    def pass1(adj):
        return pl.pallas_call(
            _p1_body,
            grid=grid_a,
            in_specs=[_row_spec(_BR, n), _full_spec((n, 64)),
                      _full_spec((64, 32))],
            out_specs=[_row_spec(_BR, 32), _row_spec(_BR, n)],
            out_shape=[jax.ShapeDtypeStruct((n, 32), BF16),
                       jax.ShapeDtypeStruct((n, n), BF16)],
            compiler_params=_PARAMS,
        )(adj, P_bf, Wg23)

    S1, A1 = pass1(adj1)
    S2, A2 = pass1(adj2)

    small16 = [_row_spec(_BR, 16), jax.ShapeDtypeStruct((n, 16), F32)]
    mu1, logvar1, mu2, logvar2, gnn_z, z, rna_emb, dec_bf, q = pl.pallas_call(
        _p2_body,
        grid=grid_a,
        in_specs=[
            _row_spec(_BR, n), _row_spec(_BR, n),
            _full_spec((n, 32)), _full_spec((n, 32)),
            _row_spec(_BR, 16),
            _full_spec(w_omega.shape), _full_spec(u_omega.shape),
            _full_spec(W_rna.shape), _full_spec((1, W_rna.shape[1])),
            _full_spec(W_dec.shape), _full_spec(cT.shape),
        ],
        out_specs=[small16[0], small16[0], small16[0], small16[0],
                   small16[0], _row_spec(_BR, 32), _row_spec(_BR, 32),
                   _row_spec(_BR, d), _row_spec(_BR, nc)],
        out_shape=[small16[1], small16[1], small16[1], small16[1],
                   small16[1],
                   jax.ShapeDtypeStruct((n, 32), F32),
                   jax.ShapeDtypeStruct((n, 32), F32),
                   jax.ShapeDtypeStruct((n, d), BF16),
                   jax.ShapeDtypeStruct((n, nc), F32)],
        compiler_params=_PARAMS,
    )(A1, A2, S1, S2, feat_x, w_omega, u_omega, W_rna,
      b_rna.reshape(1, -1), W_dec, cT)

    de_feat1, de_feat2, lp1, lp2 = pl.pallas_call(
        _p3_body,
        grid=grid_a,
        in_specs=[_row_spec(_BR, n), _row_spec(_BR, n), _full_spec((n, d)),
                  _row_spec(_BR, d), _row_spec(_BR, 1),
                  _full_spec((1, d)), _row_spec(_BR, 1)],
        out_specs=[_row_spec(_BR, d), _row_spec(_BR, d),
                   pl.BlockSpec((1, 1, 128), lambda i: (i, 0, 0)),
                   pl.BlockSpec((1, 1, 128), lambda i: (i, 0, 0))],
        out_shape=[jax.ShapeDtypeStruct((n, d), F32),
                   jax.ShapeDtypeStruct((n, d), F32),
                   jax.ShapeDtypeStruct((n // _BR, 1, 128), F32),
                   jax.ShapeDtypeStruct((n // _BR, 1, 128), F32)],
        compiler_params=_PARAMS,
    )(A1, A2, dec_bf, x, c, tok, m2)

    loss = ((jnp.sum(lp1[:, 0, 0]) + jnp.sum(lp2[:, 0, 0]))
            / num_mask).astype(F32)

    return (z, mu1, logvar1, mu2, logvar2, de_feat1, de_feat2, q,
            feat_x, gnn_z, loss, rna_emb, image_emb)


# speedup vs baseline: 1.7160x; 1.0231x over previous
import functools

import numpy as np

import jax
import jax.numpy as jnp
from jax.experimental import pallas as pl
from jax.experimental.pallas import tpu as pltpu

F32 = jnp.float32
BF16 = jnp.bfloat16

_HI = jax.lax.Precision.HIGHEST
_INV_STD = float(1.0 / (1.0 + 0.001) ** 0.5)

_BR = 400
_BRS = 2000


@functools.lru_cache(maxsize=None)
def _mask_vectors_host(n):
    num_mask = int(0.8 * n)
    with jax.ensure_compile_time_eval():
        p1 = np.asarray(jax.random.permutation(jax.random.key(1), n))
        p2 = np.asarray(jax.random.permutation(jax.random.key(2), n))
    m1 = np.zeros((n, 1), np.float32)
    m2 = np.zeros((n, 1), np.float32)
    m1[p1[:num_mask]] = 1.0
    m2[p2[:num_mask]] = 1.0
    return m1, m2


def _mask_vectors(n):
    num_mask = int(0.8 * n)
    try:
        return _mask_vectors_host(n)
    except Exception:
        p1 = jax.random.permutation(jax.random.key(1), n)
        p2 = jax.random.permutation(jax.random.key(2), n)
        m1 = jnp.zeros((n, 1), F32).at[p1[:num_mask], 0].set(1.0)
        m2 = jnp.zeros((n, 1), F32).at[p2[:num_mask], 0].set(1.0)
        return m1, m2


def _dot(a, b, precision=None):
    return jax.lax.dot_general(a, b, (((1,), (0,)), ((), ())),
                               preferred_element_type=F32,
                               precision=precision)


def _elu(v):
    return jnp.where(v > 0, v, jnp.exp(v) - 1.0)


def _enc_body(x_ref, img_ref, c_ref, tok_ref, W1_ref, s1_ref, o1_ref,
              W2_ref, s2_ref, o2_ref, Wg1_ref, Wimg_ref, bimg_ref,
              feat_ref, P_ref, iemb_ref):
    xe = x_ref[...] + c_ref[...] * tok_ref[...]
    h = _dot(xe.astype(BF16), W1_ref[...])
    h = _elu(h * s1_ref[...] + o1_ref[...])
    f = _dot(h.astype(BF16), W2_ref[...])
    f = _elu(f * s2_ref[...] + o2_ref[...])
    feat_ref[...] = f
    P_ref[...] = _dot(f.astype(BF16), Wg1_ref[...]).astype(BF16)
    iemb_ref[...] = _dot(img_ref[...].astype(BF16), Wimg_ref[...]) + bimg_ref[...]


def _p1_body(adj_ref, P_ref, Wg23_ref, S_ref, abf_ref):
    ab = adj_ref[...].astype(BF16)
    abf_ref[...] = ab
    t = _dot(ab, P_ref[...])
    h = jnp.maximum(t, 0.0)
    S_ref[...] = _dot(h.astype(BF16), Wg23_ref[...]).astype(BF16)


def _p23_body(nb, a1_ref, a2_ref, S1_ref, S2_ref, f_ref, w_ref, u_ref,
              Wr_ref, br_ref, Wd_ref, cT_ref, x_ref, c_ref, tok_ref, m2_ref,
              mu1_ref, lv1_ref, mu2_ref, lv2_ref,
              gnn_ref, z_ref, rna_ref, q_ref,
              de1_ref, de2_ref, lp1_ref, lp2_ref, dec_scr):
    i = pl.program_id(0)
    br = a1_ref.shape[0]

    @pl.when(i < nb)
    def _phase_a():
        ml1 = _dot(a1_ref[...], S1_ref[...])
        ml2 = _dot(a2_ref[...], S2_ref[...])
        mu1 = ml1[:, :16]
        mu2 = ml2[:, :16]
        mu1_ref[...] = mu1
        lv1_ref[...] = ml1[:, 16:]
        mu2_ref[...] = mu2
        lv2_ref[...] = ml2[:, 16:]
        v1 = jnp.tanh(_dot(mu1, w_ref[...], precision=_HI))
        v2 = jnp.tanh(_dot(mu2, w_ref[...], precision=_HI))
        s1 = _dot(v1, u_ref[...], precision=_HI)
        s2 = _dot(v2, u_ref[...], precision=_HI)
        w1 = jax.nn.sigmoid(s1 - s2)
        gnn = w1 * mu1 + (1.0 - w1) * mu2
        gnn_ref[...] = gnn
        z = jnp.concatenate([f_ref[...], gnn], axis=1)
        z_ref[...] = z
        rna = _dot(z, Wr_ref[...], precision=_HI) + br_ref[...]
        rna_ref[...] = rna
        dec_scr[pl.ds(i * br, br), :] = _dot(
            rna, Wd_ref[...], precision=_HI).astype(BF16)
        cT = cT_ref[...]
        zn2 = jnp.sum(z * z, axis=1, keepdims=True)
        cn2 = jnp.sum(cT * cT, axis=0, keepdims=True)
        cross = _dot(z, cT, precision=_HI)
        qd = 1.0 / (1.0 + (zn2 - 2.0 * cross + cn2))
        q_ref[...] = qd / jnp.sum(qd, axis=1, keepdims=True)

    @pl.when(i >= nb)
    def _phase_b():
        dec = dec_scr[...]
        xe = x_ref[...] + c_ref[...] * tok_ref[...]
        xn = jnp.maximum(jnp.sqrt(jnp.sum(xe * xe, axis=1, keepdims=True)),
                         1e-12)
        m2 = m2_ref[...]

        def sce(de, lp_ref):
            dn = jnp.maximum(
                jnp.sqrt(jnp.sum(de * de, axis=1, keepdims=True)), 1e-12)
            cos = jnp.sum(de * xe, axis=1, keepdims=True) / (dn * xn)
            t = 1.0 - cos
            lp_ref[...] = jnp.broadcast_to(jnp.sum(m2 * t * t * t),
                                           (1, 1, 128))

        de1 = _dot(a1_ref[...], dec)
        de1_ref[...] = de1
        sce(de1, lp1_ref)
        de2 = _dot(a2_ref[...], dec)
        de2_ref[...] = de2
        sce(de2, lp2_ref)


def _full_spec(shape):
    nd = len(shape)
    return pl.BlockSpec(shape, lambda i: (0,) * nd)


def _row_spec(br, ncols):
    return pl.BlockSpec((br, ncols), lambda i: (i, 0))


_PARAMS = pltpu.CompilerParams(dimension_semantics=("parallel",))


def kernel(x, image_feature, adj1, adj2, W_enc1, b_enc1, g_enc1, be_enc1,
           W_enc2, b_enc2, g_enc2, be_enc2, W_gc1, W_gc2, W_gc3, W_dec,
           w_omega, u_omega, W_rna, b_rna, W_img, b_img, cluster,
           enc_mask_token):
    n, d = x.shape
    img_d = image_feature.shape[1]
    num_mask = int(0.8 * n)

    m1, m2 = _mask_vectors(n)
    c = jnp.asarray(m1 + m2)
    m2 = jnp.asarray(m2)
    tok = enc_mask_token.reshape(1, d)

    s1 = (_INV_STD * g_enc1).reshape(1, -1)
    o1 = (b_enc1 * _INV_STD * g_enc1 + be_enc1).reshape(1, -1)
    s2 = (_INV_STD * g_enc2).reshape(1, -1)
    o2 = (b_enc2 * _INV_STD * g_enc2 + be_enc2).reshape(1, -1)
    Wg23 = jnp.concatenate([W_gc2, W_gc3], axis=1).astype(BF16)
    cT = cluster.T
    nc = cluster.shape[0]

    grid_s = (n // _BRS,)
    feat_x, P_bf, image_emb = pl.pallas_call(
        _enc_body,
        grid=grid_s,
        in_specs=[
            _row_spec(_BRS, d), _row_spec(_BRS, img_d), _row_spec(_BRS, 1),
            _full_spec((1, d)), _full_spec(W_enc1.shape), _full_spec(s1.shape),
            _full_spec(o1.shape), _full_spec(W_enc2.shape),
            _full_spec(s2.shape), _full_spec(o2.shape),
            _full_spec(W_gc1.shape), _full_spec(W_img.shape),
            _full_spec((1, W_img.shape[1])),
        ],
        out_specs=[_row_spec(_BRS, 16), _row_spec(_BRS, 64),
                   _row_spec(_BRS, 32)],
        out_shape=[jax.ShapeDtypeStruct((n, 16), F32),
                   jax.ShapeDtypeStruct((n, 64), BF16),
                   jax.ShapeDtypeStruct((n, 32), F32)],
        compiler_params=_PARAMS,
    )(x, image_feature, c, tok, W_enc1.astype(BF16), s1, o1,
      W_enc2.astype(BF16), s2, o2, W_gc1.astype(BF16),
      W_img.astype(BF16), b_img.reshape(1, -1))

    grid_a = (n // _BR,)

    def pass1(adj):
        return pl.pallas_call(
            _p1_body,
            grid=grid_a,
            in_specs=[_row_spec(_BR, n), _full_spec((n, 64)),
                      _full_spec((64, 32))],
            out_specs=[_row_spec(_BR, 32), _row_spec(_BR, n)],
            out_shape=[jax.ShapeDtypeStruct((n, 32), BF16),
                       jax.ShapeDtypeStruct((n, n), BF16)],
            compiler_params=_PARAMS,
        )(adj, P_bf, Wg23)

    S1, A1 = pass1(adj1)
    S2, A2 = pass1(adj2)

    nb = n // _BR

    def _mod_spec(ncols):
        return pl.BlockSpec((_BR, ncols), lambda i: (i % nb, 0))

    def _pa_spec(ncols):
        return pl.BlockSpec((_BR, ncols), lambda i: (jnp.minimum(i, nb - 1), 0))

    def _pb_spec(ncols):
        return pl.BlockSpec((_BR, ncols), lambda i: (jnp.maximum(i - nb, 0), 0))

    _lp_spec = pl.BlockSpec((1, 1, 128), lambda i: (jnp.maximum(i - nb, 0), 0, 0))
    s16 = jax.ShapeDtypeStruct((n, 16), F32)
    s32 = jax.ShapeDtypeStruct((n, 32), F32)

    (mu1, logvar1, mu2, logvar2, gnn_z, z, rna_emb, q,
     de_feat1, de_feat2, lp1, lp2) = pl.pallas_call(
        functools.partial(_p23_body, nb),
        grid=(2 * nb,),
        in_specs=[
            _mod_spec(n), _mod_spec(n),
            _full_spec((n, 32)), _full_spec((n, 32)),
            _pa_spec(16),
            _full_spec(w_omega.shape), _full_spec(u_omega.shape),
            _full_spec(W_rna.shape), _full_spec((1, W_rna.shape[1])),
            _full_spec(W_dec.shape), _full_spec(cT.shape),
            _pb_spec(d), _pb_spec(1), _full_spec((1, d)), _pb_spec(1),
        ],
        out_specs=[_pa_spec(16), _pa_spec(16), _pa_spec(16), _pa_spec(16),
                   _pa_spec(16), _pa_spec(32), _pa_spec(32), _pa_spec(nc),
                   _pb_spec(d), _pb_spec(d), _lp_spec, _lp_spec],
        out_shape=[s16, s16, s16, s16, s16, s32, s32,
                   jax.ShapeDtypeStruct((n, nc), F32),
                   jax.ShapeDtypeStruct((n, d), F32),
                   jax.ShapeDtypeStruct((n, d), F32),
                   jax.ShapeDtypeStruct((nb, 1, 128), F32),
                   jax.ShapeDtypeStruct((nb, 1, 128), F32)],
        scratch_shapes=[pltpu.VMEM((n, d), BF16)],
        compiler_params=pltpu.CompilerParams(
            dimension_semantics=("arbitrary",)),
    )(A1, A2, S1, S2, feat_x, w_omega, u_omega, W_rna,
      b_rna.reshape(1, -1), W_dec, cT, x, c, tok, m2)

    loss = ((jnp.sum(lp1[:, 0, 0]) + jnp.sum(lp2[:, 0, 0]))
            / num_mask).astype(F32)

    return (z, mu1, logvar1, mu2, logvar2, de_feat1, de_feat2, q,
            feat_x, gnn_z, loss, rna_emb, image_emb)
